# baseline (speedup 1.0000x reference)
import jax
import jax.numpy as jnp
from jax import lax
from jax.experimental import pallas as pl
from jax.experimental.pallas import tpu as pltpu
from jax.experimental.pallas import tpu_sc as plsc

NC, NS, L = 2, 16, 16
NW = NC * NS
B = 16384
F = 32
BPW = B // NW
CHUNK = 128
NCH = BPW // CHUNK


def _sc_body(user_hbm, item_hbm, ut_hbm, it_hbm, out_hbm,
             uidx_v, iidx_v, urows_v, irows_v, out_v, sem):
    wid = lax.axis_index("s") * NC + lax.axis_index("c")
    base = wid * BPW

    pltpu.sync_copy(user_hbm.at[pl.ds(base, BPW)], uidx_v)
    pltpu.sync_copy(item_hbm.at[pl.ds(base, BPW)], iidx_v)

    copies = []
    for c in range(NCH):
        sl = pl.ds(c * CHUNK, CHUNK)
        copies.append(pltpu.async_copy(ut_hbm.at[uidx_v.at[sl]], urows_v.at[sl], sem))
        copies.append(pltpu.async_copy(it_hbm.at[iidx_v.at[sl]], irows_v.at[sl], sem))
    for cp in copies:
        cp.wait()

    lanes = lax.iota(jnp.int32, L)

    def group(g, carry):
        row0 = g * L
        rows = row0 + lanes
        acc = jnp.zeros((L,), jnp.float32)
        for f in range(F):
            col = jnp.full((L,), f, jnp.int32)
            u = plsc.load_gather(urows_v, [rows, col])
            v = plsc.load_gather(irows_v, [rows, col])
            acc = acc + u * v
        out_v[pl.ds(row0, L)] = acc
        return carry

    lax.fori_loop(0, BPW // L, group, 0)

    pltpu.sync_copy(out_v, out_hbm.at[pl.ds(base, BPW)])


def kernel(user, item, user_table, item_table):
    mesh = plsc.VectorSubcoreMesh(core_axis_name="c", subcore_axis_name="s")
    k = pl.kernel(
        _sc_body,
        out_type=jax.ShapeDtypeStruct((B,), jnp.float32),
        mesh=mesh,
        compiler_params=pltpu.CompilerParams(
            needs_layout_passes=False, use_tc_tiling_on_sc=False),
        scratch_types=[
            pltpu.VMEM((BPW,), jnp.int32),
            pltpu.VMEM((BPW,), jnp.int32),
            pltpu.VMEM((BPW, F), jnp.float32),
            pltpu.VMEM((BPW, F), jnp.float32),
            pltpu.VMEM((BPW,), jnp.float32),
            pltpu.SemaphoreType.DMA,
        ],
    )
    return k(user, item, user_table, item_table)
